# Initial kernel scaffold; baseline (speedup 1.0000x reference)
import jax
import jax.numpy as jnp
from jax.experimental import pallas as pl


def kernel(x, edges, W_gcn, b_gcn, W_lin, b_lin):
    raise NotImplementedError("write your pallas kernel here")


# speedup vs baseline: 41.5928x; 41.5928x over previous
import functools
import jax
import jax.numpy as jnp
from jax import lax
from jax.experimental import pallas as pl
from jax.experimental.pallas import tpu as pltpu, tpu_sc as plsc

N = 10000
E = 320000
DIN = 128
NC = 2
NS = 16
NW = NC * NS
CHW = 128
NCH = 79
EPT = NCH * CHW
EPAD = NW * EPT
NPAD = 10112
SLICE = NPAD // NS

_mesh = plsc.VectorSubcoreMesh(core_axis_name="c", subcore_axis_name="s")


@functools.partial(
    pl.kernel,
    out_type=jax.ShapeDtypeStruct((NC * NPAD,), jnp.float32),
    mesh=_mesh,
    scratch_types=[
        pltpu.VMEM((NCH, CHW), jnp.int32),
        pltpu.VMEM((CHW,), jnp.float32),
        pltpu.VMEM((SLICE,), jnp.float32),
        pltpu.VMEM_SHARED((NPAD,), jnp.float32),
    ],
)
def _sc_degree(col_hbm, ones_hbm, zeros_hbm, cnt_hbm, col_v, ones_v, buf_v,
               shared):
    c = lax.axis_index("c")
    s = lax.axis_index("s")
    wid = c * NS + s
    pltpu.sync_copy(col_hbm.at[wid], col_v)
    pltpu.sync_copy(ones_hbm, ones_v)
    pltpu.sync_copy(zeros_hbm.at[pl.ds(0, SLICE)], buf_v)
    pltpu.sync_copy(buf_v, shared.at[pl.ds(s * SLICE, SLICE)])
    plsc.subcore_barrier()

    def step(j, carry):
        pltpu.sync_copy(ones_v, shared.at[col_v.at[j]], add=True)
        return carry

    lax.fori_loop(0, NCH, step, 0)
    plsc.subcore_barrier()
    pltpu.sync_copy(shared.at[pl.ds(s * SLICE, SLICE)], buf_v)
    pltpu.sync_copy(buf_v, cnt_hbm.at[pl.ds(c * NPAD + s * SLICE, SLICE)])


@functools.partial(
    pl.kernel,
    out_type=jax.ShapeDtypeStruct((NC * 3 * NPAD,), jnp.float32),
    mesh=_mesh,
    scratch_types=[
        pltpu.VMEM((NCH, CHW), jnp.int32),
        pltpu.VMEM((NCH, CHW), jnp.int32),
        pltpu.VMEM((CHW,), jnp.float32),
        pltpu.VMEM((CHW,), jnp.float32),
        pltpu.VMEM((CHW,), jnp.float32),
        pltpu.VMEM((SLICE,), jnp.float32),
        pltpu.VMEM_SHARED((NPAD,), jnp.float32),
        pltpu.VMEM_SHARED((NPAD,), jnp.float32),
        pltpu.VMEM_SHARED((NPAD,), jnp.float32),
        pltpu.SemaphoreType.DMA,
        pltpu.SemaphoreType.DMA,
        pltpu.SemaphoreType.DMA,
    ],
)
def _sc_scatter(row_hbm, col_hbm, y0_hbm, y1_hbm, y2_hbm, zeros_hbm, acc_hbm,
                row_v, col_v, m0, m1, m2, buf_v, sh0, sh1, sh2,
                sem0, sem1, sem2):
    c = lax.axis_index("c")
    s = lax.axis_index("s")
    wid = c * NS + s
    pltpu.sync_copy(row_hbm.at[wid], row_v)
    pltpu.sync_copy(col_hbm.at[wid], col_v)
    pltpu.sync_copy(zeros_hbm.at[pl.ds(0, SLICE)], buf_v)
    for sh in (sh0, sh1, sh2):
        pltpu.sync_copy(buf_v, sh.at[pl.ds(s * SLICE, SLICE)])
    plsc.subcore_barrier()

    def step(j, carry):
        idx = row_v.at[j]
        d0 = pltpu.async_copy(y0_hbm.at[idx], m0, sem0)
        d1 = pltpu.async_copy(y1_hbm.at[idx], m1, sem1)
        d2 = pltpu.async_copy(y2_hbm.at[idx], m2, sem2)
        d0.wait()
        d1.wait()
        d2.wait()
        cdx = col_v.at[j]
        pltpu.sync_copy(m0, sh0.at[cdx], add=True)
        pltpu.sync_copy(m1, sh1.at[cdx], add=True)
        pltpu.sync_copy(m2, sh2.at[cdx], add=True)
        return carry

    lax.fori_loop(0, NCH, step, 0)
    plsc.subcore_barrier()
    for d, sh in enumerate((sh0, sh1, sh2)):
        pltpu.sync_copy(sh.at[pl.ds(s * SLICE, SLICE)], buf_v)
        pltpu.sync_copy(
            buf_v, acc_hbm.at[pl.ds((c * 3 + d) * NPAD + s * SLICE, SLICE)])


def _tc_prep_body(cnt_ref, x_ref, w_ref, y_ref, dinv_ref):
    cnt = cnt_ref[...]
    deg = cnt[0:1] + cnt[1:2] + 1.0
    dinv = lax.rsqrt(deg)
    dinv_ref[...] = dinv
    xw_t = lax.dot_general(w_ref[...], x_ref[...],
                           dimension_numbers=(((0,), (1,)), ((), ())),
                           preferred_element_type=jnp.float32)
    y = xw_t * dinv[:, :N]
    y_ref[...] = jnp.concatenate(
        [y, jnp.zeros((3, NPAD - N), jnp.float32)], axis=1)


_tc_prep = pl.pallas_call(
    _tc_prep_body,
    out_shape=[
        jax.ShapeDtypeStruct((3, NPAD), jnp.float32),
        jax.ShapeDtypeStruct((1, NPAD), jnp.float32),
    ],
)


def _tc_final_body(acc_ref, y_ref, dinv_ref, bg_ref, wlt_ref, bl_ref,
                   ht_ref, zt_ref):
    acc = acc_ref[...]
    y = y_ref[...]
    dinv = dinv_ref[...]
    a = acc[0:3] + acc[3:6]
    agg = dinv * (a + y)
    h_t = jnp.maximum(agg + bg_ref[...], 0.0)
    ht_ref[...] = h_t
    wlt = wlt_ref[...]
    z_t = (wlt[:, 0:1] * h_t[0:1] + wlt[:, 1:2] * h_t[1:2]
           + wlt[:, 2:3] * h_t[2:3] + bl_ref[...])
    zt_ref[...] = z_t


_tc_final = pl.pallas_call(
    _tc_final_body,
    out_shape=[
        jax.ShapeDtypeStruct((3, NPAD), jnp.float32),
        jax.ShapeDtypeStruct((4, NPAD), jnp.float32),
    ],
)


def kernel(x, edges, W_gcn, b_gcn, W_lin, b_lin):
    pad = EPAD - E
    rowp = jnp.concatenate([edges[0], jnp.zeros((pad,), jnp.int32)])
    colp = jnp.concatenate([edges[1], jnp.full((pad,), N, jnp.int32)])
    rowp = rowp.reshape(NW, NCH, CHW)
    colp = colp.reshape(NW, NCH, CHW)

    ones128 = jnp.ones((CHW,), jnp.float32)
    zeros1 = jnp.zeros((NPAD,), jnp.float32)

    cnt = _sc_degree(colp, ones128, zeros1)
    y_all, dinv = _tc_prep(cnt.reshape(NC, NPAD), x, W_gcn)
    acc = _sc_scatter(rowp, colp, y_all[0], y_all[1], y_all[2],
                      zeros1)

    h_t, z_t = _tc_final(acc.reshape(NC * 3, NPAD), y_all, dinv,
                         b_gcn.reshape(3, 1), W_lin.T, b_lin.reshape(4, 1))
    return (h_t[:, :N].T, z_t[:, :N].T)
